# baseline (speedup 1.0000x reference)
import functools

import jax
import jax.numpy as jnp
from jax import lax
from jax.experimental import pallas as pl
from jax.experimental.pallas import tpu as pltpu
from jax.experimental.pallas import tpu_sc as plsc

_N = 10000
_E = 160000
_D = 256
_H = 128
_NT = 16
_EPT = _E // _NT
_CH = 80
_NCH = _EPT // _CH
_CHP = 80
_RPT = 632
_NPAD = 10240
_DEGR = _NPAD // _NT
_RB = 400


def _sc_deg_body(dst3, invout, deg_s, dst_v, ones_v, invb, repl):
    c = lax.axis_index("c")
    s = lax.axis_index("s")

    @pl.when(c == 0)
    def _():
        pltpu.sync_copy(dst3.at[s], dst_v)

        def _zero(i, carry):
            invb[pl.ds(i * 16, 16)] = jnp.zeros((16,), jnp.float32)
            return carry
        lax.fori_loop(0, _DEGR // 16, _zero, 0)

        def _one(i, carry):
            ones_v[pl.ds(i * 16, 16)] = jnp.full((16,), 1.0, jnp.float32)
            return carry
        lax.fori_loop(0, _CHP // 16, _one, 0)

        pltpu.sync_copy(invb, deg_s.at[pl.ds(s * _DEGR, _DEGR)])

    plsc.subcore_barrier()

    @pl.when(c == 0)
    def _():
        def _chunk(j, carry):
            pltpu.sync_copy(ones_v.at[pl.ds(0, _CH)],
                            deg_s.at[dst_v.at[j]], add=True)
            return carry
        lax.fori_loop(0, _NCH, _chunk, 0)

    plsc.subcore_barrier()

    @pl.when(c == 0)
    def _():
        pltpu.sync_copy(deg_s.at[pl.ds(s * _DEGR, _DEGR)], invb)

        def _inv(i, carry):
            v = invb[pl.ds(i * 16, 16)]
            invb[pl.ds(i * 16, 16)] = 1.0 / (v + 1.0)
            return carry
        lax.fori_loop(0, _DEGR // 16, _inv, 0)

        def _repl(i, carry):
            v = invb[pl.ds(i * 16, 16)]
            for k in range(16):
                repl[i * 16 + k] = jnp.full((16,), 1.0, jnp.float32) * v[k]
            return carry
        lax.fori_loop(0, _DEGR // 16, _repl, 0)
        pltpu.sync_copy(repl, invout.at[pl.ds(s * _DEGR, _DEGR)])


_sc_deg = pl.kernel(
    _sc_deg_body,
    out_type=[jax.ShapeDtypeStruct((_NPAD, 16), jnp.float32)],
    mesh=plsc.VectorSubcoreMesh(core_axis_name="c", subcore_axis_name="s"),
    scratch_types=[
        pltpu.VMEM_SHARED((_NPAD,), jnp.float32),
        pltpu.VMEM((_NCH, _CH), jnp.int32),
        pltpu.VMEM((_CHP,), jnp.float32),
        pltpu.VMEM((_DEGR,), jnp.float32),
        pltpu.VMEM((_DEGR, 16), jnp.float32),
    ],
)


def _sc_agg_body(ytab, srcb, dstb, aggy, agg_s, src_v, dst_v, buf, *sems):
    gsems = sems[:3]
    ssems = sems[3:]
    c = lax.axis_index("c")
    s = lax.axis_index("s")

    pltpu.sync_copy(srcb.at[c, s], src_v)
    pltpu.sync_copy(dstb.at[s], dst_v)

    _LAST = _N - 15 * _RPT

    @pl.when(s < 15)
    def _():
        pltpu.sync_copy(
            ytab.at[pl.ds(c * _N + s * _RPT, _RPT)],
            agg_s.at[pl.ds(s * _RPT, _RPT)],
        )

    @pl.when(s == 15)
    def _():
        pltpu.sync_copy(
            ytab.at[pl.ds(c * _N + 15 * _RPT, _LAST)],
            agg_s.at[pl.ds(15 * _RPT, _LAST)],
        )

    plsc.subcore_barrier()

    bufs = [buf.at[b] for b in range(3)]

    def _eidx(v, j):
        return v.at[pl.ds(j * _CH, _CH)]

    def _start_g(j, b):
        pltpu.async_copy(ytab.at[_eidx(src_v, j)], bufs[b], gsems[b])

    def _wait_g(j, b):
        pltpu.make_async_copy(ytab.at[_eidx(src_v, j)], bufs[b],
                              gsems[b]).wait()

    def _start_s(j, b):
        return
        pltpu.async_copy(bufs[b], agg_s.at[_eidx(dst_v, j)], ssems[b],
                         add=True)

    def _wait_s(j, b):
        return
        pltpu.make_async_copy(bufs[b], agg_s.at[_eidx(dst_v, j)],
                              ssems[b]).wait()

    _start_g(0, 0)
    _start_g(1, 1)

    def _tri(o, carry):
        for b in range(3):
            j = 3 * o + b
            b2 = (b + 2) % 3
            _wait_g(j, b)

            @pl.when(j >= 1)
            def _():
                _wait_s(j - 1, b2)
            _start_g(j + 2, b2)
            _start_s(j, b)
        return carry
    lax.fori_loop(0, (_NCH - 2) // 3, _tri, 0)

    _wait_g(_NCH - 2, (_NCH - 2) % 3)
    _start_s(_NCH - 2, (_NCH - 2) % 3)
    _wait_g(_NCH - 1, (_NCH - 1) % 3)
    _start_s(_NCH - 1, (_NCH - 1) % 3)
    _wait_s(_NCH - 3, (_NCH - 3) % 3)
    _wait_s(_NCH - 2, (_NCH - 2) % 3)
    _wait_s(_NCH - 1, (_NCH - 1) % 3)

    plsc.subcore_barrier()

    @pl.when(s < 15)
    def _():
        pltpu.sync_copy(
            agg_s.at[pl.ds(s * _RPT, _RPT)],
            aggy.at[c, pl.ds(s * _RPT, _RPT)],
        )

    @pl.when(s == 15)
    def _():
        pltpu.sync_copy(
            agg_s.at[pl.ds(15 * _RPT, _LAST)],
            aggy.at[c, pl.ds(15 * _RPT, _LAST)],
        )


_sc_agg = pl.kernel(
    _sc_agg_body,
    out_type=[jax.ShapeDtypeStruct((2, _N, _H), jnp.float32)],
    mesh=plsc.VectorSubcoreMesh(core_axis_name="c", subcore_axis_name="s"),
    scratch_types=[
        pltpu.VMEM_SHARED((_N, _H), jnp.float32),
        pltpu.VMEM((_EPT,), jnp.int32),
        pltpu.VMEM((_EPT,), jnp.int32),
        pltpu.VMEM((3, _CH, _H), jnp.float32),
    ] + [pltpu.SemaphoreType.DMA] * 6,
)


def _mm1_body(x_ref, w_ref, y_ref):
    y = jnp.dot(x_ref[...], w_ref[...], preferred_element_type=jnp.float32)
    y_ref[0] = y[:, :_H]
    y_ref[1] = y[:, _H:]


def _mm2_body(agg_ref, inv_ref, b_ref, w_ref, y_ref):
    a = jnp.concatenate([agg_ref[0], agg_ref[1]], axis=1)
    h = jnp.maximum(a * inv_ref[:, 0:1] + b_ref[0], 0.0)
    y = jnp.dot(h, w_ref[...], preferred_element_type=jnp.float32)
    y_ref[0] = y[:, :_H]
    y_ref[1] = y[:, _H:]


def _ep_body(agg_ref, inv_ref, b_ref, o_ref):
    a = jnp.concatenate([agg_ref[0], agg_ref[1]], axis=1)
    o_ref[...] = a * inv_ref[:, 0:1] + b_ref[0]


def _mm1(x, w):
    return pl.pallas_call(
        _mm1_body,
        grid=(_N // _RB,),
        in_specs=[
            pl.BlockSpec((_RB, _D), lambda r: (r, 0)),
            pl.BlockSpec((_D, _D), lambda r: (0, 0)),
        ],
        out_specs=pl.BlockSpec((2, _RB, _H), lambda r: (0, r, 0)),
        out_shape=jax.ShapeDtypeStruct((2, _N, _H), jnp.float32),
    )(x, w)


def _mm2(aggy, inv, b, w):
    return pl.pallas_call(
        _mm2_body,
        grid=(_N // _RB,),
        in_specs=[
            pl.BlockSpec((2, _RB, _H), lambda r: (0, r, 0)),
            pl.BlockSpec((_RB, 16), lambda r: (r, 0)),
            pl.BlockSpec((1, _D), lambda r: (0, 0)),
            pl.BlockSpec((_D, _D), lambda r: (0, 0)),
        ],
        out_specs=pl.BlockSpec((2, _RB, _H), lambda r: (0, r, 0)),
        out_shape=jax.ShapeDtypeStruct((2, _N, _H), jnp.float32),
    )(aggy, inv, b, w)


def _ep(aggy, inv, b):
    return pl.pallas_call(
        _ep_body,
        grid=(_N // _RB,),
        in_specs=[
            pl.BlockSpec((2, _RB, _H), lambda r: (0, r, 0)),
            pl.BlockSpec((_RB, 16), lambda r: (r, 0)),
            pl.BlockSpec((1, _D), lambda r: (0, 0)),
        ],
        out_specs=pl.BlockSpec((_RB, _D), lambda r: (r, 0)),
        out_shape=jax.ShapeDtypeStruct((_N, _D), jnp.float32),
    )(aggy, inv, b)


def kernel(features, edge_index, W1, b1, W2, b2):
    src = edge_index[0]
    dst = edge_index[1]
    srcb = jnp.stack([src, src + _N]).reshape(2, _NT, _EPT)
    dstb = dst.reshape(_NT, _EPT)
    dst3 = dst.reshape(_NT, _NCH, _CH)
    b1r = b1.reshape(1, _D)
    b2r = b2.reshape(1, _D)

    inv, = _sc_deg(dst3)
    y1 = _mm1(features, W1)
    aggy1, = _sc_agg(y1.reshape(2 * _N, _H), srcb, dstb)
    y2 = _mm2(aggy1, inv, b1r, W2)
    aggy2, = _sc_agg(y2.reshape(2 * _N, _H), srcb, dstb)
    return _ep(aggy2, inv, b2r)


# speedup vs baseline: 1.0431x; 1.0431x over previous
import functools

import jax
import jax.numpy as jnp
from jax import lax
from jax.experimental import pallas as pl
from jax.experimental.pallas import tpu as pltpu
from jax.experimental.pallas import tpu_sc as plsc

_N = 10000
_E = 160000
_D = 256
_H = 128
_NT = 16
_EPT = _E // _NT
_CH = 40
_NCH = _EPT // _CH
_NB = 6
_LOOK = 4
_DCH = 80
_DNCH = _EPT // _DCH
_RPT = 632
_NPAD = 10240
_DEGR = _NPAD // _NT
_RB = 400


def _sc_deg_body(dst3, invout, deg_s, dst_v, ones_v, invb, repl):
    c = lax.axis_index("c")
    s = lax.axis_index("s")

    @pl.when(c == 0)
    def _():
        pltpu.sync_copy(dst3.at[s], dst_v)

        def _zero(i, carry):
            invb[pl.ds(i * 16, 16)] = jnp.zeros((16,), jnp.float32)
            return carry
        lax.fori_loop(0, _DEGR // 16, _zero, 0)

        def _one(i, carry):
            ones_v[pl.ds(i * 16, 16)] = jnp.full((16,), 1.0, jnp.float32)
            return carry
        lax.fori_loop(0, _DCH // 16, _one, 0)

        pltpu.sync_copy(invb, deg_s.at[pl.ds(s * _DEGR, _DEGR)])

    plsc.subcore_barrier()

    @pl.when(c == 0)
    def _():
        def _chunk(j, carry):
            pltpu.sync_copy(ones_v, deg_s.at[dst_v.at[j]], add=True)
            return carry
        lax.fori_loop(0, _DNCH, _chunk, 0)

    plsc.subcore_barrier()

    @pl.when(c == 0)
    def _():
        pltpu.sync_copy(deg_s.at[pl.ds(s * _DEGR, _DEGR)], invb)

        def _inv(i, carry):
            v = invb[pl.ds(i * 16, 16)]
            invb[pl.ds(i * 16, 16)] = 1.0 / (v + 1.0)
            return carry
        lax.fori_loop(0, _DEGR // 16, _inv, 0)

        def _repl(i, carry):
            v = invb[pl.ds(i * 16, 16)]
            for k in range(16):
                repl[i * 16 + k] = jnp.full((16,), 1.0, jnp.float32) * v[k]
            return carry
        lax.fori_loop(0, _DEGR // 16, _repl, 0)
        pltpu.sync_copy(repl, invout.at[pl.ds(s * _DEGR, _DEGR)])


_sc_deg = pl.kernel(
    _sc_deg_body,
    out_type=[jax.ShapeDtypeStruct((_NPAD, 16), jnp.float32)],
    mesh=plsc.VectorSubcoreMesh(core_axis_name="c", subcore_axis_name="s"),
    scratch_types=[
        pltpu.VMEM_SHARED((_NPAD,), jnp.float32),
        pltpu.VMEM((_DNCH, _DCH), jnp.int32),
        pltpu.VMEM((_DCH,), jnp.float32),
        pltpu.VMEM((_DEGR,), jnp.float32),
        pltpu.VMEM((_DEGR, 16), jnp.float32),
    ],
)


def _sc_agg_body(ytab, srcb, dstb, aggy, agg_s, src_v, dst_v, buf, *sems):
    gsems = sems[:_NB]
    ssems = sems[_NB:]
    c = lax.axis_index("c")
    s = lax.axis_index("s")

    pltpu.sync_copy(srcb.at[c, s], src_v)
    pltpu.sync_copy(dstb.at[s], dst_v)

    _LAST = _N - 15 * _RPT

    @pl.when(s < 15)
    def _():
        pltpu.sync_copy(
            ytab.at[pl.ds(c * _N + s * _RPT, _RPT)],
            agg_s.at[pl.ds(s * _RPT, _RPT)],
        )

    @pl.when(s == 15)
    def _():
        pltpu.sync_copy(
            ytab.at[pl.ds(c * _N + 15 * _RPT, _LAST)],
            agg_s.at[pl.ds(15 * _RPT, _LAST)],
        )

    plsc.subcore_barrier()

    bufs = [buf.at[b] for b in range(_NB)]

    def _eidx(v, j):
        return v.at[pl.ds(j * _CH, _CH)]

    def _start_g(j, b):
        pltpu.async_copy(ytab.at[_eidx(src_v, j)], bufs[b], gsems[b])

    def _wait_g(j, b):
        pltpu.make_async_copy(ytab.at[_eidx(src_v, j)], bufs[b],
                              gsems[b]).wait()

    def _start_s(j, b):
        pltpu.async_copy(bufs[b], agg_s.at[_eidx(dst_v, j)], ssems[b],
                         add=True)

    def _wait_s(j, b):
        pltpu.make_async_copy(bufs[b], agg_s.at[_eidx(dst_v, j)],
                              ssems[b]).wait()

    for b in range(_LOOK):
        _start_g(b, b)

    def _step(j, b):
        b2 = (b + _LOOK) % _NB
        _wait_g(j, b)

        @pl.when(j >= _NB - _LOOK)
        def _():
            _wait_s(j - (_NB - _LOOK), b2)
        _start_g(j + _LOOK, b2)
        _start_s(j, b)

    def _sext(o, carry):
        for b in range(_NB):
            _step(_NB * o + b, b)
        return carry
    lax.fori_loop(0, (_NCH - _LOOK) // _NB, _sext, 0)

    for j in range(_NCH - _LOOK, _NCH):
        b = j % _NB
        _wait_g(j, b)
        _wait_s(j - (_NB - _LOOK), (b + _LOOK) % _NB)
        _start_s(j, b)
    for j in range(_NCH - _NB + _LOOK, _NCH):
        _wait_s(j, j % _NB)

    plsc.subcore_barrier()

    @pl.when(s < 15)
    def _():
        pltpu.sync_copy(
            agg_s.at[pl.ds(s * _RPT, _RPT)],
            aggy.at[c, pl.ds(s * _RPT, _RPT)],
        )

    @pl.when(s == 15)
    def _():
        pltpu.sync_copy(
            agg_s.at[pl.ds(15 * _RPT, _LAST)],
            aggy.at[c, pl.ds(15 * _RPT, _LAST)],
        )


_sc_agg = pl.kernel(
    _sc_agg_body,
    out_type=[jax.ShapeDtypeStruct((2, _N, _H), jnp.float32)],
    mesh=plsc.VectorSubcoreMesh(core_axis_name="c", subcore_axis_name="s"),
    scratch_types=[
        pltpu.VMEM_SHARED((_N, _H), jnp.float32),
        pltpu.VMEM((_EPT,), jnp.int32),
        pltpu.VMEM((_EPT,), jnp.int32),
        pltpu.VMEM((_NB, _CH, _H), jnp.float32),
    ] + [pltpu.SemaphoreType.DMA] * (2 * _NB),
)


def _mm1_body(x_ref, w_ref, y_ref):
    y = jnp.dot(x_ref[...], w_ref[...], preferred_element_type=jnp.float32)
    y_ref[0] = y[:, :_H]
    y_ref[1] = y[:, _H:]


def _mm2_body(agg_ref, inv_ref, b_ref, w_ref, y_ref):
    a = jnp.concatenate([agg_ref[0], agg_ref[1]], axis=1)
    h = jnp.maximum(a * inv_ref[:, 0:1] + b_ref[0], 0.0)
    y = jnp.dot(h, w_ref[...], preferred_element_type=jnp.float32)
    y_ref[0] = y[:, :_H]
    y_ref[1] = y[:, _H:]


def _ep_body(agg_ref, inv_ref, b_ref, o_ref):
    a = jnp.concatenate([agg_ref[0], agg_ref[1]], axis=1)
    o_ref[...] = a * inv_ref[:, 0:1] + b_ref[0]


def _mm1(x, w):
    return pl.pallas_call(
        _mm1_body,
        grid=(_N // _RB,),
        in_specs=[
            pl.BlockSpec((_RB, _D), lambda r: (r, 0)),
            pl.BlockSpec((_D, _D), lambda r: (0, 0)),
        ],
        out_specs=pl.BlockSpec((2, _RB, _H), lambda r: (0, r, 0)),
        out_shape=jax.ShapeDtypeStruct((2, _N, _H), jnp.float32),
    )(x, w)


def _mm2(aggy, inv, b, w):
    return pl.pallas_call(
        _mm2_body,
        grid=(_N // _RB,),
        in_specs=[
            pl.BlockSpec((2, _RB, _H), lambda r: (0, r, 0)),
            pl.BlockSpec((_RB, 16), lambda r: (r, 0)),
            pl.BlockSpec((1, _D), lambda r: (0, 0)),
            pl.BlockSpec((_D, _D), lambda r: (0, 0)),
        ],
        out_specs=pl.BlockSpec((2, _RB, _H), lambda r: (0, r, 0)),
        out_shape=jax.ShapeDtypeStruct((2, _N, _H), jnp.float32),
    )(aggy, inv, b, w)


def _ep(aggy, inv, b):
    return pl.pallas_call(
        _ep_body,
        grid=(_N // _RB,),
        in_specs=[
            pl.BlockSpec((2, _RB, _H), lambda r: (0, r, 0)),
            pl.BlockSpec((_RB, 16), lambda r: (r, 0)),
            pl.BlockSpec((1, _D), lambda r: (0, 0)),
        ],
        out_specs=pl.BlockSpec((_RB, _D), lambda r: (r, 0)),
        out_shape=jax.ShapeDtypeStruct((_N, _D), jnp.float32),
    )(aggy, inv, b)


def kernel(features, edge_index, W1, b1, W2, b2):
    src = edge_index[0]
    dst = edge_index[1]
    srcb = jnp.stack([src, src + _N]).reshape(2, _NT, _EPT)
    dstb = dst.reshape(_NT, _EPT)
    dst3 = dst.reshape(_NT, _DNCH, _DCH)
    b1r = b1.reshape(1, _D)
    b2r = b2.reshape(1, _D)

    inv, = _sc_deg(dst3)
    y1 = _mm1(features, W1)
    aggy1, = _sc_agg(y1.reshape(2 * _N, _H), srcb, dstb)
    y2 = _mm2(aggy1, inv, b1r, W2)
    aggy2, = _sc_agg(y2.reshape(2 * _N, _H), srcb, dstb)
    return _ep(aggy2, inv, b2r)


# speedup vs baseline: 1.1193x; 1.0730x over previous
import functools

import jax
import jax.numpy as jnp
from jax import lax
from jax.experimental import pallas as pl
from jax.experimental.pallas import tpu as pltpu
from jax.experimental.pallas import tpu_sc as plsc

_N = 10000
_E = 160000
_D = 256
_H = 128
_NT = 16
_EPT = _E // _NT
_CH = 40
_NCH = _EPT // _CH
_NB = 6
_LOOK = 4
_DCH = 80
_DNCH = _EPT // _DCH
_RPT = 632
_NPAD = 10240
_DEGR = _NPAD // _NT
_RB = 1000


def _sc_deg_body(dst3, invout, deg_s, dst_v, ones_v, invb, repl):
    c = lax.axis_index("c")
    s = lax.axis_index("s")

    @pl.when(c == 0)
    def _():
        pltpu.sync_copy(dst3.at[s], dst_v)

        def _zero(i, carry):
            invb[pl.ds(i * 16, 16)] = jnp.zeros((16,), jnp.float32)
            return carry
        lax.fori_loop(0, _DEGR // 16, _zero, 0)

        def _one(i, carry):
            ones_v[pl.ds(i * 16, 16)] = jnp.full((16,), 1.0, jnp.float32)
            return carry
        lax.fori_loop(0, _DCH // 16, _one, 0)

        pltpu.sync_copy(invb, deg_s.at[pl.ds(s * _DEGR, _DEGR)])

    plsc.subcore_barrier()

    @pl.when(c == 0)
    def _():
        def _chunk(j, carry):
            pltpu.sync_copy(ones_v, deg_s.at[dst_v.at[j]], add=True)
            return carry
        lax.fori_loop(0, _DNCH, _chunk, 0)

    plsc.subcore_barrier()

    @pl.when(c == 0)
    def _():
        pltpu.sync_copy(deg_s.at[pl.ds(s * _DEGR, _DEGR)], invb)

        def _inv(i, carry):
            v = invb[pl.ds(i * 16, 16)]
            invb[pl.ds(i * 16, 16)] = 1.0 / (v + 1.0)
            return carry
        lax.fori_loop(0, _DEGR // 16, _inv, 0)

        def _repl(i, carry):
            v = invb[pl.ds(i * 16, 16)]
            for k in range(16):
                repl[i * 16 + k] = jnp.full((16,), 1.0, jnp.float32) * v[k]
            return carry
        lax.fori_loop(0, _DEGR // 16, _repl, 0)
        pltpu.sync_copy(repl, invout.at[pl.ds(s * _DEGR, _DEGR)])


_sc_deg = pl.kernel(
    _sc_deg_body,
    out_type=[jax.ShapeDtypeStruct((_NPAD, 16), jnp.float32)],
    mesh=plsc.VectorSubcoreMesh(core_axis_name="c", subcore_axis_name="s"),
    scratch_types=[
        pltpu.VMEM_SHARED((_NPAD,), jnp.float32),
        pltpu.VMEM((_DNCH, _DCH), jnp.int32),
        pltpu.VMEM((_DCH,), jnp.float32),
        pltpu.VMEM((_DEGR,), jnp.float32),
        pltpu.VMEM((_DEGR, 16), jnp.float32),
    ],
)


def _sc_agg_body(ytab, srcb, dstb, aggy, agg_s, src_v, dst_v, buf, *sems):
    gsems = sems[:_NB]
    ssems = sems[_NB:]
    c = lax.axis_index("c")
    s = lax.axis_index("s")

    pltpu.sync_copy(srcb.at[c, s], src_v)
    pltpu.sync_copy(dstb.at[s], dst_v)

    _LAST = _N - 15 * _RPT

    @pl.when(s < 15)
    def _():
        pltpu.sync_copy(
            ytab.at[pl.ds(c * _N + s * _RPT, _RPT)],
            agg_s.at[pl.ds(s * _RPT, _RPT)],
        )

    @pl.when(s == 15)
    def _():
        pltpu.sync_copy(
            ytab.at[pl.ds(c * _N + 15 * _RPT, _LAST)],
            agg_s.at[pl.ds(15 * _RPT, _LAST)],
        )

    plsc.subcore_barrier()

    bufs = [buf.at[b] for b in range(_NB)]

    def _eidx(v, j):
        return v.at[pl.ds(j * _CH, _CH)]

    def _start_g(j, b):
        pltpu.async_copy(ytab.at[_eidx(src_v, j)], bufs[b], gsems[b])

    def _wait_g(j, b):
        pltpu.make_async_copy(ytab.at[_eidx(src_v, j)], bufs[b],
                              gsems[b]).wait()

    def _start_s(j, b):
        pltpu.async_copy(bufs[b], agg_s.at[_eidx(dst_v, j)], ssems[b],
                         add=True)

    def _wait_s(j, b):
        pltpu.make_async_copy(bufs[b], agg_s.at[_eidx(dst_v, j)],
                              ssems[b]).wait()

    for b in range(_LOOK):
        _start_g(b, b)

    def _step(j, b):
        b2 = (b + _LOOK) % _NB
        _wait_g(j, b)

        @pl.when(j >= _NB - _LOOK)
        def _():
            _wait_s(j - (_NB - _LOOK), b2)
        _start_g(j + _LOOK, b2)
        _start_s(j, b)

    def _sext(o, carry):
        for b in range(_NB):
            _step(_NB * o + b, b)
        return carry
    lax.fori_loop(0, (_NCH - _LOOK) // _NB, _sext, 0)

    for j in range(_NCH - _LOOK, _NCH):
        b = j % _NB
        _wait_g(j, b)
        _wait_s(j - (_NB - _LOOK), (b + _LOOK) % _NB)
        _start_s(j, b)
    for j in range(_NCH - _NB + _LOOK, _NCH):
        _wait_s(j, j % _NB)

    plsc.subcore_barrier()

    @pl.when(s < 15)
    def _():
        pltpu.sync_copy(
            agg_s.at[pl.ds(s * _RPT, _RPT)],
            aggy.at[c, pl.ds(s * _RPT, _RPT)],
        )

    @pl.when(s == 15)
    def _():
        pltpu.sync_copy(
            agg_s.at[pl.ds(15 * _RPT, _LAST)],
            aggy.at[c, pl.ds(15 * _RPT, _LAST)],
        )


_sc_agg = pl.kernel(
    _sc_agg_body,
    out_type=[jax.ShapeDtypeStruct((2, _N, _H), jnp.float32)],
    mesh=plsc.VectorSubcoreMesh(core_axis_name="c", subcore_axis_name="s"),
    scratch_types=[
        pltpu.VMEM_SHARED((_N, _H), jnp.float32),
        pltpu.VMEM((_EPT,), jnp.int32),
        pltpu.VMEM((_EPT,), jnp.int32),
        pltpu.VMEM((_NB, _CH, _H), jnp.float32),
    ] + [pltpu.SemaphoreType.DMA] * (2 * _NB),
)


def _mm1_body(x_ref, w_ref, y_ref):
    y = jnp.dot(x_ref[...], w_ref[...], preferred_element_type=jnp.float32)
    y_ref[0] = y[:, :_H]
    y_ref[1] = y[:, _H:]


def _mm2_body(agg_ref, inv_ref, b_ref, w_ref, y_ref):
    a = jnp.concatenate([agg_ref[0], agg_ref[1]], axis=1)
    h = jnp.maximum(a * inv_ref[:, 0:1] + b_ref[0], 0.0)
    y = jnp.dot(h, w_ref[...], preferred_element_type=jnp.float32)
    y_ref[0] = y[:, :_H]
    y_ref[1] = y[:, _H:]


def _ep_body(agg_ref, inv_ref, b_ref, o_ref):
    a = jnp.concatenate([agg_ref[0], agg_ref[1]], axis=1)
    o_ref[...] = a * inv_ref[:, 0:1] + b_ref[0]


def _mm1(x, w):
    return pl.pallas_call(
        _mm1_body,
        grid=(_N // _RB,),
        in_specs=[
            pl.BlockSpec((_RB, _D), lambda r: (r, 0)),
            pl.BlockSpec((_D, _D), lambda r: (0, 0)),
        ],
        out_specs=pl.BlockSpec((2, _RB, _H), lambda r: (0, r, 0)),
        out_shape=jax.ShapeDtypeStruct((2, _N, _H), jnp.float32),
    )(x, w)


def _mm2(aggy, inv, b, w):
    return pl.pallas_call(
        _mm2_body,
        grid=(_N // _RB,),
        in_specs=[
            pl.BlockSpec((2, _RB, _H), lambda r: (0, r, 0)),
            pl.BlockSpec((_RB, 16), lambda r: (r, 0)),
            pl.BlockSpec((1, _D), lambda r: (0, 0)),
            pl.BlockSpec((_D, _D), lambda r: (0, 0)),
        ],
        out_specs=pl.BlockSpec((2, _RB, _H), lambda r: (0, r, 0)),
        out_shape=jax.ShapeDtypeStruct((2, _N, _H), jnp.float32),
    )(aggy, inv, b, w)


def _ep(aggy, inv, b):
    return pl.pallas_call(
        _ep_body,
        grid=(_N // _RB,),
        in_specs=[
            pl.BlockSpec((2, _RB, _H), lambda r: (0, r, 0)),
            pl.BlockSpec((_RB, 16), lambda r: (r, 0)),
            pl.BlockSpec((1, _D), lambda r: (0, 0)),
        ],
        out_specs=pl.BlockSpec((_RB, _D), lambda r: (r, 0)),
        out_shape=jax.ShapeDtypeStruct((_N, _D), jnp.float32),
    )(aggy, inv, b)


def kernel(features, edge_index, W1, b1, W2, b2):
    src = edge_index[0]
    dst = edge_index[1]
    srcb = jnp.stack([src, src + _N]).reshape(2, _NT, _EPT)
    dstb = dst.reshape(_NT, _EPT)
    dst3 = dst.reshape(_NT, _DNCH, _DCH)
    b1r = b1.reshape(1, _D)
    b2r = b2.reshape(1, _D)

    inv, = _sc_deg(dst3)
    y1 = _mm1(features, W1)
    aggy1, = _sc_agg(y1.reshape(2 * _N, _H), srcb, dstb)
    y2 = _mm2(aggy1, inv, b1r, W2)
    aggy2, = _sc_agg(y2.reshape(2 * _N, _H), srcb, dstb)
    return _ep(aggy2, inv, b2r)
